# baseline (speedup 1.0000x reference)
import jax
import jax.numpy as jnp
from jax import lax
from jax.experimental import pallas as pl
from jax.experimental.pallas import tpu as pltpu
from jax.experimental.pallas import tpu_sc as plsc

B, N, Q = 16, 2048, 128
NC, NS, L = 2, 16, 16
QH = Q // 2
NQV = QH // L
NOV = Q // L
UNROLL = 4


def _sc_body(xs_hbm, ys_hbm, samples_hbm, rho_wt_hbm, rho_b_hbm, out_hbm,
             xs_v, ys_v, samp_v, rho_v, part_v, pair_v, bias_v,
             outb_v, shared):
    c = lax.axis_index("c")
    s = lax.axis_index("s")
    b = (NS // 2) * c + s // 2
    h = s % 2

    pltpu.sync_copy(xs_hbm.at[b], xs_v)
    pltpu.sync_copy(ys_hbm.at[b], ys_v)
    pltpu.sync_copy(samples_hbm.at[pl.ds(h * QH, QH)], samp_v)
    pltpu.sync_copy(rho_wt_hbm.at[pl.ds(h * QH, QH)], rho_v)
    pltpu.sync_copy(rho_b_hbm, bias_v)

    samp = [samp_v[pl.ds(j * L, L)] for j in range(NQV)]

    def point_step(i, acc):
        base = i * L
        xv = xs_v[pl.ds(base, L)]
        yv = ys_v[pl.ds(base, L)]
        for u in range(L):
            x = xv[u]
            y = yv[u]
            acc = tuple(
                a + jnp.maximum(jnp.minimum(t - x, y - t), 0.0)
                for a, t in zip(acc, samp)
            )
        return acc

    acc0 = tuple(jnp.zeros((L,), jnp.float32) for _ in range(NQV))
    acc = lax.fori_loop(0, N // L, point_step, acc0)

    part = [jnp.zeros((L,), jnp.float32) for _ in range(NOV)]
    for j in range(NQV):
        for u in range(L):
            w = acc[j][u]
            k = j * L + u
            part = [p + w * rho_v[k, pl.ds(j2 * L, L)]
                    for j2, p in enumerate(part)]
    for j in range(NOV):
        part_v[pl.ds(j * L, L)] = part[j]

    pltpu.sync_copy(part_v, shared.at[s])
    plsc.subcore_barrier()

    @pl.when(h == 0)
    def _():
        pltpu.sync_copy(shared.at[s + 1], pair_v)
        for j in range(NOV):
            sl = pl.ds(j * L, L)
            outb_v[sl] = jnp.maximum(part_v[sl] + pair_v[sl] + bias_v[sl],
                                     0.0)
        pltpu.sync_copy(outb_v, out_hbm.at[b])


def kernel(diagram, samples, rho_w, rho_b):
    xs = diagram[..., 0]
    ys = diagram[..., 1]
    rho_wt = rho_w.T
    fn = pl.kernel(
        _sc_body,
        out_type=jax.ShapeDtypeStruct((B, Q), jnp.float32),
        mesh=plsc.VectorSubcoreMesh(core_axis_name="c", subcore_axis_name="s",
                                    num_cores=NC, num_subcores=NS),
        scratch_types=[
            pltpu.VMEM((N,), jnp.float32),
            pltpu.VMEM((N,), jnp.float32),
            pltpu.VMEM((QH,), jnp.float32),
            pltpu.VMEM((QH, Q), jnp.float32),
            pltpu.VMEM((Q,), jnp.float32),
            pltpu.VMEM((Q,), jnp.float32),
            pltpu.VMEM((Q,), jnp.float32),
            pltpu.VMEM((Q,), jnp.float32),
            pltpu.VMEM_SHARED((NS, Q), jnp.float32),
        ],
    )
    return fn(xs, ys, samples, rho_wt, rho_b)


# speedup vs baseline: 1.2867x; 1.2867x over previous
import jax
import jax.numpy as jnp
from jax import lax
from jax.experimental import pallas as pl
from jax.experimental.pallas import tpu as pltpu
from jax.experimental.pallas import tpu_sc as plsc

B, N, Q = 16, 2048, 128
NC, NS, L = 2, 16, 16
QH = Q // 2
NG = QH // 4
N_SC = 1024
N_TC = N - N_SC
CTC = N_TC // 128


def _sc_body(xs_hbm, ys_hbm, samples_hbm, out_hbm, xs_v, ys_v, samp_v,
             accs_v):
    c = lax.axis_index("c")
    s = lax.axis_index("s")
    b = (NS // 2) * c + s // 2
    h = s % 2

    pltpu.sync_copy(xs_hbm.at[b, pl.ds(0, N_SC)], xs_v)
    pltpu.sync_copy(ys_hbm.at[b, pl.ds(0, N_SC)], ys_v)
    pltpu.sync_copy(samples_hbm.at[pl.ds(h * QH, QH)], samp_v.at[pl.ds(0, QH)])

    def group_step(g, carry):
        sv = samp_v[pl.ds(4 * g, L)]
        t = [jnp.zeros((L,), jnp.float32) + sv[j] for j in range(4)]

        def point_step(i, acc):
            for u in range(4):
                base = i * (4 * L) + u * L
                xv = xs_v[pl.ds(base, L)]
                yv = ys_v[pl.ds(base, L)]
                acc = tuple(
                    a + jnp.maximum(jnp.minimum(tj - xv, yv - tj), 0.0)
                    for a, tj in zip(acc, t)
                )
            return acc

        acc0 = tuple(jnp.zeros((L,), jnp.float32) for _ in range(4))
        acc = lax.fori_loop(0, N_SC // (4 * L), point_step, acc0)
        for j in range(4):
            accs_v[pl.ds((4 * g + j) * L, L)] = acc[j]
        return carry

    lax.fori_loop(0, NG, group_step, jnp.int32(0))

    pltpu.sync_copy(accs_v, out_hbm.at[b, h])


def _tc_pool_body(xs_ref, ys_ref, samp_ref, out_ref):
    sg = [samp_ref[g] for g in range(16)]
    accs = [jnp.zeros((8, 128), jnp.float32) for _ in range(16)]
    for c in range(CTC):
        xb = jnp.broadcast_to(xs_ref[0, c][None, :], (8, 128))
        yb = jnp.broadcast_to(ys_ref[0, c][None, :], (8, 128))
        for g in range(16):
            accs[g] = accs[g] + jnp.maximum(
                jnp.minimum(sg[g] - xb, yb - sg[g]), 0.0)
    out_ref[0] = jnp.stack([jnp.sum(a, axis=1) for a in accs], axis=0)


def _tc_combine_body(z0_ref, z1_ref, w0_ref, w1_ref, tc_ref, w_ref, b_ref,
                     out_ref):
    hi = lax.Precision.HIGHEST
    z = lax.dot_general(z0_ref[...], w0_ref[...], (((1,), (0,)), ((), ())),
                        precision=hi, preferred_element_type=jnp.float32)
    z = z + lax.dot_general(z1_ref[...], w1_ref[...], (((1,), (0,)), ((), ())),
                            precision=hi, preferred_element_type=jnp.float32)
    z = z + lax.dot_general(tc_ref[...], w_ref[...], (((1,), (1,)), ((), ())),
                            precision=hi, preferred_element_type=jnp.float32)
    out_ref[...] = jnp.maximum(z + b_ref[...], 0.0)


def kernel(diagram, samples, rho_w, rho_b):
    xs = diagram[..., 0]
    ys = diagram[..., 1]

    accs_sc = pl.kernel(
        _sc_body,
        out_type=jax.ShapeDtypeStruct((B, 2, QH * L), jnp.float32),
        mesh=plsc.VectorSubcoreMesh(core_axis_name="c", subcore_axis_name="s",
                                    num_cores=NC, num_subcores=NS),
        scratch_types=[
            pltpu.VMEM((N_SC,), jnp.float32),
            pltpu.VMEM((N_SC,), jnp.float32),
            pltpu.VMEM((QH + L,), jnp.float32),
            pltpu.VMEM((QH * L,), jnp.float32),
        ],
    )(xs, ys, samples)

    xs_tc = xs[:, N_SC:].reshape(B, CTC, 128)
    ys_tc = ys[:, N_SC:].reshape(B, CTC, 128)
    samples_bc = jnp.broadcast_to(samples.reshape(16, 8, 1), (16, 8, 128))

    pooled_tc = pl.pallas_call(
        _tc_pool_body,
        grid=(B,),
        in_specs=[
            pl.BlockSpec((1, CTC, 128), lambda b: (b, 0, 0)),
            pl.BlockSpec((1, CTC, 128), lambda b: (b, 0, 0)),
            pl.BlockSpec((16, 8, 128), lambda b: (0, 0, 0)),
        ],
        out_specs=pl.BlockSpec((1, 16, 8), lambda b: (b, 0, 0)),
        out_shape=jax.ShapeDtypeStruct((B, 16, 8), jnp.float32),
    )(xs_tc, ys_tc, samples_bc).reshape(B, Q)

    z0 = accs_sc[:, 0, :]
    z1 = accs_sc[:, 1, :]
    wt = rho_w.T
    w0 = jnp.broadcast_to(wt[:QH, None, :], (QH, L, Q)).reshape(QH * L, Q)
    w1 = jnp.broadcast_to(wt[QH:, None, :], (QH, L, Q)).reshape(QH * L, Q)

    out = pl.pallas_call(
        _tc_combine_body,
        in_specs=[
            pl.BlockSpec((B, QH * L), lambda: (0, 0)),
            pl.BlockSpec((B, QH * L), lambda: (0, 0)),
            pl.BlockSpec((QH * L, Q), lambda: (0, 0)),
            pl.BlockSpec((QH * L, Q), lambda: (0, 0)),
            pl.BlockSpec((B, Q), lambda: (0, 0)),
            pl.BlockSpec((Q, Q), lambda: (0, 0)),
            pl.BlockSpec((1, Q), lambda: (0, 0)),
        ],
        out_specs=pl.BlockSpec((B, Q), lambda: (0, 0)),
        out_shape=jax.ShapeDtypeStruct((B, Q), jnp.float32),
    )(z0, z1, w0, w1, pooled_tc, rho_w, rho_b.reshape(1, Q))
    return out


# speedup vs baseline: 1.3627x; 1.0590x over previous
import jax
import jax.numpy as jnp
from jax import lax
from jax.experimental import pallas as pl
from jax.experimental.pallas import tpu as pltpu
from jax.experimental.pallas import tpu_sc as plsc

B, N, Q = 16, 2048, 128
NC, NS, L = 2, 16, 16
QH = Q // 2
NG = QH // 4
N_SC = 1024
N_TC = N - N_SC
CTC = N_TC // 128


def _sc_body(xs_hbm, ys_hbm, samples_hbm, out_hbm, xs_v, ys_v, samp_v,
             accs_v):
    c = lax.axis_index("c")
    s = lax.axis_index("s")
    b = (NS // 2) * c + s // 2
    h = s % 2

    pltpu.sync_copy(xs_hbm.at[b, pl.ds(0, N_SC)], xs_v)
    pltpu.sync_copy(ys_hbm.at[b, pl.ds(0, N_SC)], ys_v)
    pltpu.sync_copy(samples_hbm.at[pl.ds(h * QH, QH)], samp_v.at[pl.ds(0, QH)])

    def group_step(g, carry):
        sv = samp_v[pl.ds(4 * g, L)]
        t = [jnp.zeros((L,), jnp.float32) + sv[j] for j in range(4)]

        def point_step(i, acc):
            for u in range(4):
                base = i * (4 * L) + u * L
                xv = xs_v[pl.ds(base, L)]
                yv = ys_v[pl.ds(base, L)]
                acc = tuple(
                    a + jnp.maximum(jnp.minimum(tj - xv, yv - tj), 0.0)
                    for a, tj in zip(acc, t)
                )
            return acc

        acc0 = tuple(jnp.zeros((L,), jnp.float32) for _ in range(4))
        acc = lax.fori_loop(0, N_SC // (4 * L), point_step, acc0)
        for j in range(4):
            accs_v[pl.ds((4 * g + j) * L, L)] = acc[j]
        return carry

    lax.fori_loop(0, NG, group_step, jnp.int32(0))

    pltpu.sync_copy(accs_v, out_hbm.at[b, h])


def _tc_pool_body(xs_ref, ys_ref, samp_ref, out_ref):
    rows = []
    for g in range(16):
        sg = samp_ref[g]
        acc = jnp.zeros((8, 128), jnp.float32)
        for c in range(CTC):
            xb = jnp.broadcast_to(xs_ref[0, c][None, :], (8, 128))
            yb = jnp.broadcast_to(ys_ref[0, c][None, :], (8, 128))
            acc = acc + jnp.maximum(jnp.minimum(sg - xb, yb - sg), 0.0)
        rows.append(jnp.sum(acc, axis=1))
    out_ref[0] = jnp.stack(rows, axis=0)


def _tc_combine_body(zc_ref, r_ref, tc_ref, w_ref, b_ref, out_ref):
    hi = lax.Precision.HIGHEST
    pooled_sc = lax.dot_general(zc_ref[...], r_ref[...],
                                (((1,), (0,)), ((), ())),
                                precision=hi,
                                preferred_element_type=jnp.float32)
    pooled = pooled_sc + tc_ref[...]
    z = lax.dot_general(pooled, w_ref[...], (((1,), (1,)), ((), ())),
                        precision=hi, preferred_element_type=jnp.float32)
    out_ref[...] = jnp.maximum(z + b_ref[...], 0.0)


def kernel(diagram, samples, rho_w, rho_b):
    xs = diagram[..., 0]
    ys = diagram[..., 1]

    accs_sc = pl.kernel(
        _sc_body,
        out_type=jax.ShapeDtypeStruct((B, 2, QH * L), jnp.float32),
        mesh=plsc.VectorSubcoreMesh(core_axis_name="c", subcore_axis_name="s",
                                    num_cores=NC, num_subcores=NS),
        scratch_types=[
            pltpu.VMEM((N_SC,), jnp.float32),
            pltpu.VMEM((N_SC,), jnp.float32),
            pltpu.VMEM((QH + L,), jnp.float32),
            pltpu.VMEM((QH * L,), jnp.float32),
        ],
    )(xs, ys, samples)

    xs_tc = xs[:, N_SC:].reshape(B, CTC, 128)
    ys_tc = ys[:, N_SC:].reshape(B, CTC, 128)
    samples_bc = jnp.broadcast_to(samples.reshape(16, 8, 1), (16, 8, 128))

    pooled_tc = pl.pallas_call(
        _tc_pool_body,
        grid=(B,),
        in_specs=[
            pl.BlockSpec((1, CTC, 128), lambda b: (b, 0, 0)),
            pl.BlockSpec((1, CTC, 128), lambda b: (b, 0, 0)),
            pl.BlockSpec((16, 8, 128), lambda b: (0, 0, 0)),
        ],
        out_specs=pl.BlockSpec((1, 16, 8), lambda b: (b, 0, 0)),
        out_shape=jax.ShapeDtypeStruct((B, 16, 8), jnp.float32),
    )(xs_tc, ys_tc, samples_bc).reshape(B, Q)

    zc = accs_sc.reshape(B, 2 * QH * L)
    rows_k = jnp.arange(2 * QH * L, dtype=jnp.int32) // L
    r_mat = (rows_k[:, None] == jnp.arange(Q, dtype=jnp.int32)[None, :]
             ).astype(jnp.float32)

    out = pl.pallas_call(
        _tc_combine_body,
        in_specs=[
            pl.BlockSpec((B, 2 * QH * L), lambda: (0, 0)),
            pl.BlockSpec((2 * QH * L, Q), lambda: (0, 0)),
            pl.BlockSpec((B, Q), lambda: (0, 0)),
            pl.BlockSpec((Q, Q), lambda: (0, 0)),
            pl.BlockSpec((1, Q), lambda: (0, 0)),
        ],
        out_specs=pl.BlockSpec((B, Q), lambda: (0, 0)),
        out_shape=jax.ShapeDtypeStruct((B, Q), jnp.float32),
    )(zc, r_mat, pooled_tc, rho_w, rho_b.reshape(1, Q))
    return out


# speedup vs baseline: 1.8039x; 1.3238x over previous
import jax
import jax.numpy as jnp
import numpy as np
from jax import lax
from jax.experimental import pallas as pl
from jax.experimental.pallas import tpu as pltpu
from jax.experimental.pallas import tpu_sc as plsc

B, N, Q = 16, 2048, 128
NC, NS, L = 2, 16, 16
QH = Q // 2
NG = QH // 4
N_SC = 1024
N_TC = N - N_SC
CTC = N_TC // 128


def _sc_body(xs_hbm, ys_hbm, samples_hbm, out_hbm, xs_v, ys_v, samp_v,
             accs_v):
    c = lax.axis_index("c")
    s = lax.axis_index("s")
    b = (NS // 2) * c + s // 2
    h = s % 2

    pltpu.sync_copy(xs_hbm.at[b, pl.ds(0, N_SC)], xs_v)
    pltpu.sync_copy(ys_hbm.at[b, pl.ds(0, N_SC)], ys_v)
    pltpu.sync_copy(samples_hbm.at[pl.ds(h * QH, QH)], samp_v.at[pl.ds(0, QH)])

    def group_step(g, carry):
        sv = samp_v[pl.ds(4 * g, L)]
        t = [jnp.zeros((L,), jnp.float32) + sv[j] for j in range(4)]

        def point_step(i, acc):
            for u in range(4):
                base = i * (4 * L) + u * L
                xv = xs_v[pl.ds(base, L)]
                yv = ys_v[pl.ds(base, L)]
                acc = tuple(
                    a + jnp.maximum(jnp.minimum(tj - xv, yv - tj), 0.0)
                    for a, tj in zip(acc, t)
                )
            return acc

        acc0 = tuple(jnp.zeros((L,), jnp.float32) for _ in range(4))
        acc = lax.fori_loop(0, N_SC // (4 * L), point_step, acc0)
        for j in range(4):
            accs_v[pl.ds((4 * g + j) * L, L)] = acc[j]
        return carry

    lax.fori_loop(0, NG, group_step, jnp.int32(0))

    pltpu.sync_copy(accs_v, out_hbm.at[b, pl.ds(h * QH * L, QH * L)])


def _tc_pool_body(xs_ref, ys_ref, samp_ref, out_ref):
    for bi in range(8):
        rows = []
        for g in range(16):
            sg = samp_ref[g]
            acc = jnp.zeros((8, 128), jnp.float32)
            for c in range(CTC):
                xr = xs_ref[bi, pl.ds(c * 128, 128)]
                yr = ys_ref[bi, pl.ds(c * 128, 128)]
                xb = jnp.broadcast_to(xr[None, :], (8, 128))
                yb = jnp.broadcast_to(yr[None, :], (8, 128))
                acc = acc + jnp.maximum(jnp.minimum(sg - xb, yb - sg), 0.0)
            rows.append(jnp.sum(acc, axis=1))
        out_ref[bi, :] = jnp.concatenate(rows, axis=0)


def _tc_combine_body(zc_ref, r_ref, tc_ref, w_ref, b_ref, out_ref):
    hi = lax.Precision.HIGHEST
    pooled_sc = lax.dot_general(zc_ref[...], r_ref[...],
                                (((1,), (0,)), ((), ())),
                                precision=hi,
                                preferred_element_type=jnp.float32)
    pooled = pooled_sc + tc_ref[...]
    z = lax.dot_general(pooled, w_ref[...], (((1,), (1,)), ((), ())),
                        precision=hi, preferred_element_type=jnp.float32)
    out_ref[...] = jnp.maximum(z + b_ref[...], 0.0)


def kernel(diagram, samples, rho_w, rho_b):
    xs = diagram[..., 0]
    ys = diagram[..., 1]

    accs_sc = pl.kernel(
        _sc_body,
        out_type=jax.ShapeDtypeStruct((B, 2 * QH * L), jnp.float32),
        mesh=plsc.VectorSubcoreMesh(core_axis_name="c", subcore_axis_name="s",
                                    num_cores=NC, num_subcores=NS),
        scratch_types=[
            pltpu.VMEM((N_SC,), jnp.float32),
            pltpu.VMEM((N_SC,), jnp.float32),
            pltpu.VMEM((QH + L,), jnp.float32),
            pltpu.VMEM((QH * L,), jnp.float32),
        ],
    )(xs, ys, samples)

    xs_tc = xs[:, N_SC:]
    ys_tc = ys[:, N_SC:]
    samples_bc = jnp.broadcast_to(samples.reshape(16, 8, 1), (16, 8, 128))

    pooled_tc = pl.pallas_call(
        _tc_pool_body,
        grid=(B // 8,),
        in_specs=[
            pl.BlockSpec((8, N_TC), lambda b: (b, 0)),
            pl.BlockSpec((8, N_TC), lambda b: (b, 0)),
            pl.BlockSpec((16, 8, 128), lambda b: (0, 0, 0)),
        ],
        out_specs=pl.BlockSpec((8, Q), lambda b: (b, 0)),
        out_shape=jax.ShapeDtypeStruct((B, Q), jnp.float32),
    )(xs_tc, ys_tc, samples_bc)

    r_mat = jnp.asarray(np.equal.outer(np.arange(2 * QH * L) // L,
                                       np.arange(Q)).astype(np.float32))

    out = pl.pallas_call(
        _tc_combine_body,
        in_specs=[
            pl.BlockSpec((B, 2 * QH * L), lambda: (0, 0)),
            pl.BlockSpec((2 * QH * L, Q), lambda: (0, 0)),
            pl.BlockSpec((B, Q), lambda: (0, 0)),
            pl.BlockSpec((Q, Q), lambda: (0, 0)),
            pl.BlockSpec((1, Q), lambda: (0, 0)),
        ],
        out_specs=pl.BlockSpec((B, Q), lambda: (0, 0)),
        out_shape=jax.ShapeDtypeStruct((B, Q), jnp.float32),
    )(accs_sc, r_mat, pooled_tc, rho_w, rho_b.reshape(1, Q))
    return out


# speedup vs baseline: 1.8254x; 1.0119x over previous
import jax
import jax.numpy as jnp
from jax import lax
from jax.experimental import pallas as pl
from jax.experimental.pallas import tpu as pltpu
from jax.experimental.pallas import tpu_sc as plsc

B, N, Q = 16, 2048, 128
NC, NS, L = 2, 16, 16
QH = Q // 2
NG = QH // 4
N_SC = 1024
N_TC = N - N_SC
CTC = N_TC // 128


def _sc_body(diag_hbm, samples_hbm, out_hbm, d_v, samp_v, pool_v):
    c = lax.axis_index("c")
    s = lax.axis_index("s")
    b = (NS // 2) * c + s // 2
    h = s % 2

    pltpu.sync_copy(diag_hbm.at[b, pl.ds(0, 2 * N_SC)], d_v)
    pltpu.sync_copy(samples_hbm.at[pl.ds(h * QH, QH)], samp_v)

    t = [samp_v[pl.ds(j * L, L)] for j in range(QH // L)]

    def point_step(i, acc):
        dv = d_v[pl.ds(i * L, L)]
        for u in range(8):
            x = dv[2 * u]
            y = dv[2 * u + 1]
            acc = tuple(
                a + jnp.maximum(jnp.minimum(tj - x, y - tj), 0.0)
                for a, tj in zip(acc, t)
            )
        return acc

    acc0 = tuple(jnp.zeros((L,), jnp.float32) for _ in range(QH // L))
    acc = lax.fori_loop(0, 2 * N_SC // L, point_step, acc0)
    for j in range(QH // L):
        pool_v[pl.ds(j * L, L)] = acc[j]

    pltpu.sync_copy(pool_v, out_hbm.at[b, pl.ds(h * QH, QH)])


def _tc_pool_body(xs_ref, ys_ref, samp_ref, out_ref):
    for bi in range(8):
        rows = []
        for g in range(16):
            sg = samp_ref[g]
            acc = jnp.zeros((8, 128), jnp.float32)
            for c in range(CTC):
                xr = xs_ref[bi, pl.ds(c * 128, 128)]
                yr = ys_ref[bi, pl.ds(c * 128, 128)]
                xb = jnp.broadcast_to(xr[None, :], (8, 128))
                yb = jnp.broadcast_to(yr[None, :], (8, 128))
                acc = acc + jnp.maximum(jnp.minimum(sg - xb, yb - sg), 0.0)
            rows.append(jnp.sum(acc, axis=1))
        out_ref[bi, :] = jnp.concatenate(rows, axis=0)


def _tc_combine_body(sc_ref, tc_ref, w_ref, b_ref, out_ref):
    pooled = sc_ref[...] + tc_ref[...]
    z = lax.dot_general(pooled, w_ref[...], (((1,), (1,)), ((), ())),
                        precision=lax.Precision.HIGHEST,
                        preferred_element_type=jnp.float32)
    out_ref[...] = jnp.maximum(z + b_ref[...], 0.0)


def kernel(diagram, samples, rho_w, rho_b):
    xs = diagram[..., 0]
    ys = diagram[..., 1]

    pooled_sc = pl.kernel(
        _sc_body,
        out_type=jax.ShapeDtypeStruct((B, Q), jnp.float32),
        mesh=plsc.VectorSubcoreMesh(core_axis_name="c", subcore_axis_name="s",
                                    num_cores=NC, num_subcores=NS),
        scratch_types=[
            pltpu.VMEM((2 * N_SC,), jnp.float32),
            pltpu.VMEM((QH,), jnp.float32),
            pltpu.VMEM((QH,), jnp.float32),
        ],
    )(diagram.reshape(B, 2 * N), samples)

    xs_tc = xs[:, N_SC:]
    ys_tc = ys[:, N_SC:]
    samples_bc = jnp.broadcast_to(samples.reshape(16, 8, 1), (16, 8, 128))

    pooled_tc = pl.pallas_call(
        _tc_pool_body,
        grid=(B // 8,),
        in_specs=[
            pl.BlockSpec((8, N_TC), lambda b: (b, 0)),
            pl.BlockSpec((8, N_TC), lambda b: (b, 0)),
            pl.BlockSpec((16, 8, 128), lambda b: (0, 0, 0)),
        ],
        out_specs=pl.BlockSpec((8, Q), lambda b: (b, 0)),
        out_shape=jax.ShapeDtypeStruct((B, Q), jnp.float32),
    )(xs_tc, ys_tc, samples_bc)

    out = pl.pallas_call(
        _tc_combine_body,
        in_specs=[
            pl.BlockSpec((B, Q), lambda: (0, 0)),
            pl.BlockSpec((B, Q), lambda: (0, 0)),
            pl.BlockSpec((Q, Q), lambda: (0, 0)),
            pl.BlockSpec((1, Q), lambda: (0, 0)),
        ],
        out_specs=pl.BlockSpec((B, Q), lambda: (0, 0)),
        out_shape=jax.ShapeDtypeStruct((B, Q), jnp.float32),
    )(pooled_sc, pooled_tc, rho_w, rho_b.reshape(1, Q))
    return out
